# baseline (speedup 1.0000x reference)
import jax
import jax.numpy as jnp
from jax.experimental import pallas as pl
from jax.experimental.pallas import tpu as pltpu

D_MODEL = 1024
HIDDEN = 4096
NUM_EXPERTS = 8
TOP_K = 2

TM = 512
TH = 512


def _router_body(x_ref, wg_ref, wd_ref):
    x = x_ref[...]
    wg = wg_ref[...]
    logits = jax.lax.dot_general(
        x, wg, (((1,), (1,)), ((), ())), preferred_element_type=jnp.float32)
    m = jnp.max(logits, axis=-1, keepdims=True)
    p = jnp.exp(logits - m)
    s = p / jnp.sum(p, axis=-1, keepdims=True)
    ei = jax.lax.broadcasted_iota(jnp.int32, s.shape, 1)
    E = s.shape[1]
    m1 = jnp.max(s, axis=-1, keepdims=True)
    idx1 = jnp.min(jnp.where(s == m1, ei, E), axis=-1, keepdims=True)
    oh1 = ei == idx1
    s2 = jnp.where(oh1, -jnp.inf, s)
    m2 = jnp.max(s2, axis=-1, keepdims=True)
    idx2 = jnp.min(jnp.where(s2 == m2, ei, E), axis=-1, keepdims=True)
    oh2 = ei == idx2
    denom = m1 + m2
    wd_ref[...] = (jnp.where(oh1, m1 / denom, 0.0)
                   + jnp.where(oh2, m2 / denom, 0.0))


def _ffn_body(x_ref, w1_ref, b1_ref, w2_ref, b2_ref, wd_ref, out_ref):
    e = pl.program_id(1)
    h = pl.program_id(2)
    x = x_ref[...]
    hact = jax.lax.dot_general(
        x, w1_ref[0], (((1,), (1,)), ((), ())),
        preferred_element_type=jnp.float32)
    hact = jnp.maximum(hact + b1_ref[0], 0.0)
    part = jax.lax.dot_general(
        hact, w2_ref[0], (((1,), (1,)), ((), ())),
        preferred_element_type=jnp.float32)
    part = part + (h == 0).astype(jnp.float32) * b2_ref[0]
    ei = jax.lax.broadcasted_iota(jnp.int32, wd_ref.shape, 1)
    wcol = jnp.sum(jnp.where(ei == e, wd_ref[...], 0.0), axis=1,
                   keepdims=True)
    contrib = wcol * part

    @pl.when(jnp.logical_and(e == 0, h == 0))
    def _init():
        out_ref[...] = contrib

    @pl.when(jnp.logical_not(jnp.logical_and(e == 0, h == 0)))
    def _acc():
        out_ref[...] = out_ref[...] + contrib


def kernel(x, Wg, W1, b1, W2, b2):
    Bn, Sn, D = x.shape
    T = Bn * Sn
    E, H = W1.shape[0], W1.shape[1]
    x_flat = x.reshape(T, D)
    b1r = b1.reshape(E, 1, H)
    b2r = b2.reshape(E, 1, D)

    wd = pl.pallas_call(
        _router_body,
        out_shape=jax.ShapeDtypeStruct((T, E), jnp.float32),
    )(x_flat, Wg)

    grid = (T // TM, E, H // TH)
    out = pl.pallas_call(
        _ffn_body,
        grid=grid,
        in_specs=[
            pl.BlockSpec((TM, D), lambda t, e, h: (t, 0)),
            pl.BlockSpec((1, TH, D), lambda t, e, h: (e, h, 0)),
            pl.BlockSpec((1, 1, TH), lambda t, e, h: (e, 0, h)),
            pl.BlockSpec((1, D, TH), lambda t, e, h: (e, 0, h)),
            pl.BlockSpec((1, 1, D), lambda t, e, h: (e, 0, 0)),
            pl.BlockSpec((TM, E), lambda t, e, h: (t, 0)),
        ],
        out_specs=pl.BlockSpec((TM, D), lambda t, e, h: (t, 0)),
        out_shape=jax.ShapeDtypeStruct((T, D), jnp.float32),
    )(x_flat, W1, b1r, W2, b2r, wd)

    return out.reshape(Bn, Sn, D)


# speedup vs baseline: 1.5149x; 1.5149x over previous
import functools

import jax
import jax.numpy as jnp
from jax import lax
from jax.experimental import pallas as pl
from jax.experimental.pallas import tpu as pltpu
from jax.experimental.pallas import tpu_sc as plsc

D_MODEL = 1024
HIDDEN = 4096
NUM_EXPERTS = 8
T_TOK = 2048

BLK = 256
NB = 2 * T_TOK // BLK + NUM_EXPERTS
PAD = NB * BLK
TH3 = 1024
HT = HIDDEN // TH3

SC_CORES = 2
SC_SUBCORES = 16
NW = SC_CORES * SC_SUBCORES
A_PER_W = 2 * T_TOK // NW
T_PER_W = T_TOK // NW


def _router_body(x_ref, wg_ref, pos_ref, w01_ref, be_ref, nu_ref):
    T = T_TOK
    E = NUM_EXPERTS
    x = x_ref[...]
    wg = wg_ref[...]
    logits = lax.dot_general(
        x, wg, (((1,), (1,)), ((), ())), preferred_element_type=jnp.float32)
    m = jnp.max(logits, axis=-1, keepdims=True)
    p = jnp.exp(logits - m)
    s = p / jnp.sum(p, axis=-1, keepdims=True)
    ei = lax.broadcasted_iota(jnp.int32, s.shape, 1)
    m1 = jnp.max(s, axis=-1, keepdims=True)
    idx1 = jnp.min(jnp.where(s == m1, ei, E), axis=-1, keepdims=True)
    oh1 = (ei == idx1).astype(jnp.float32)
    s2 = jnp.where(oh1 > 0, -jnp.inf, s)
    m2 = jnp.max(s2, axis=-1, keepdims=True)
    idx2 = jnp.min(jnp.where(s2 == m2, ei, E), axis=-1, keepdims=True)
    oh2 = (ei == idx2).astype(jnp.float32)
    denom = m1 + m2
    w0 = m1 / denom
    w1 = m2 / denom

    oh1_3 = oh1.reshape(16, 128, E)
    oh2_3 = oh2.reshape(16, 128, E)
    ii = lax.broadcasted_iota(jnp.int32, (16, 128, 128), 1)
    jj = lax.broadcasted_iota(jnp.int32, (16, 128, 128), 2)
    tril = (ii >= jj).astype(jnp.float32)
    cw1 = lax.dot_general(tril, oh1_3, (((2,), (1,)), ((0,), (0,))),
                          preferred_element_type=jnp.float32)
    cw2 = lax.dot_general(tril, oh2_3, (((2,), (1,)), ((0,), (0,))),
                          preferred_element_type=jnp.float32)
    tot1 = cw1[:, 127, :]
    tot2 = cw2[:, 127, :]
    ci = lax.broadcasted_iota(jnp.int32, (16, 16), 0)
    cj = lax.broadcasted_iota(jnp.int32, (16, 16), 1)
    l16 = (ci > cj).astype(jnp.float32)
    pre1 = lax.dot_general(l16, tot1, (((1,), (0,)), ((), ())),
                           preferred_element_type=jnp.float32)
    pre2 = lax.dot_general(l16, tot2, (((1,), (0,)), ((), ())),
                           preferred_element_type=jnp.float32)
    cex1 = cw1 + pre1.reshape(16, 1, E) - oh1_3
    cex2 = cw2 + pre2.reshape(16, 1, E) - oh2_3

    count1_row = jnp.sum(oh1, axis=0, keepdims=True)
    counts_row = count1_row + jnp.sum(oh2, axis=0, keepdims=True)
    nbk_row = jnp.floor((counts_row + (BLK - 1.0)) * (1.0 / BLK))
    e8i = lax.broadcasted_iota(jnp.int32, (E, E), 0)
    e8j = lax.broadcasted_iota(jnp.int32, (E, E), 1)
    l8t = (e8i < e8j).astype(jnp.float32)
    po_row = BLK * lax.dot_general(nbk_row, l8t, (((1,), (0,)), ((), ())),
                                   preferred_element_type=jnp.float32)
    po_b = po_row.reshape(1, 1, E)
    c1_b = count1_row.reshape(1, 1, E)

    pos0 = jnp.sum(oh1_3 * (cex1 + po_b), axis=2)
    pos1 = jnp.sum(oh2_3 * (cex2 + po_b + c1_b), axis=2)
    pos_ref[...] = jnp.concatenate(
        [pos0, pos1], axis=0).astype(jnp.int32)

    w01_ref[0] = jnp.broadcast_to(w0, (T, 16))
    w01_ref[1] = jnp.broadcast_to(w1, (T, 16))

    idm = (e8i == e8j).astype(jnp.float32)
    po_col = lax.dot_general(idm, po_row, (((0,), (1,)), ((), ())),
                             preferred_element_type=jnp.float32)
    nb24 = lax.broadcasted_iota(jnp.int32, (1, NB), 1).astype(jnp.float32)
    cmp = (po_col * (1.0 / BLK) <= nb24).astype(jnp.float32)
    be = jnp.sum(cmp, axis=0, keepdims=True) - 1.0
    ei_row = lax.broadcasted_iota(jnp.int32, (1, E), 1)
    last_e = jnp.max(jnp.where(counts_row > 0, ei_row, -1),
                     axis=1, keepdims=True).astype(jnp.float32)
    be_ref[...] = jnp.minimum(be, last_e).astype(jnp.int32)
    nu_ref[...] = jnp.sum(nbk_row, axis=1, keepdims=True).astype(jnp.int32)


def _run_router(x_flat, Wg):
    return pl.pallas_call(
        _router_body,
        out_shape=(
            jax.ShapeDtypeStruct((NW, A_PER_W), jnp.int32),
            jax.ShapeDtypeStruct((2, T_TOK, 16), jnp.float32),
            jax.ShapeDtypeStruct((1, NB), jnp.int32),
            jax.ShapeDtypeStruct((1, 1), jnp.int32),
        ),
    )(x_flat, Wg)


def _sc_mesh():
    return plsc.VectorSubcoreMesh(core_axis_name="c", subcore_axis_name="s")


_K2_CH = 32
_K2_NCH = A_PER_W // _K2_CH


def _k2_body(x_hbm, pos_hbm, xs_hbm, idx_v, rows_v, sem):
    wid = lax.axis_index("s") * SC_CORES + lax.axis_index("c")
    pltpu.sync_copy(pos_hbm.at[wid], idx_v)
    tok0 = (wid % 16) * A_PER_W
    for j in range(_K2_NCH):
        pltpu.sync_copy(x_hbm.at[pl.ds(tok0 + j * _K2_CH, _K2_CH)], rows_v)
        pltpu.async_copy(rows_v, xs_hbm.at[idx_v.at[j]], sem).wait()


def _run_scatter(x_flat, pos3):
    k = functools.partial(
        pl.kernel, mesh=_sc_mesh(),
        out_type=jax.ShapeDtypeStruct((PAD, D_MODEL), jnp.float32),
        scratch_types=[
            pltpu.VMEM((_K2_NCH, _K2_CH), jnp.int32),
            pltpu.VMEM((_K2_CH, D_MODEL), jnp.float32),
            pltpu.SemaphoreType.DMA,
        ],
    )(_k2_body)
    return k(x_flat, pos3)


def _ffn_body(be_ref, xs_ref, w1_ref, b1_ref, w2_ref, b2_ref, ys_ref):
    h = pl.program_id(1)
    x = xs_ref[...]
    hact = lax.dot_general(
        x, w1_ref[0], (((1,), (1,)), ((), ())),
        preferred_element_type=jnp.float32)
    hact = jnp.maximum(hact + b1_ref[0], 0.0)
    part = lax.dot_general(
        hact, w2_ref[0], (((1,), (1,)), ((), ())),
        preferred_element_type=jnp.float32)

    @pl.when(h == 0)
    def _init():
        ys_ref[...] = part + b2_ref[0]

    @pl.when(h != 0)
    def _acc():
        ys_ref[...] = ys_ref[...] + part


def _run_ffn(xs, W1, b1r, W2, b2r, be24):
    grid_spec = pltpu.PrefetchScalarGridSpec(
        num_scalar_prefetch=1,
        grid=(NB, HT),
        in_specs=[
            pl.BlockSpec((BLK, D_MODEL), lambda nb, h, be: (nb, 0)),
            pl.BlockSpec((1, TH3, D_MODEL), lambda nb, h, be: (be[nb], h, 0)),
            pl.BlockSpec((1, 1, TH3), lambda nb, h, be: (be[nb], 0, h)),
            pl.BlockSpec((1, D_MODEL, TH3), lambda nb, h, be: (be[nb], 0, h)),
            pl.BlockSpec((1, 1, D_MODEL), lambda nb, h, be: (be[nb], 0, 0)),
        ],
        out_specs=pl.BlockSpec((BLK, D_MODEL), lambda nb, h, be: (nb, 0)),
    )
    return pl.pallas_call(
        _ffn_body,
        grid_spec=grid_spec,
        out_shape=jax.ShapeDtypeStruct((PAD, D_MODEL), jnp.float32),
    )(be24, xs, W1, b1r, W2, b2r)


_K4_CH = 32


def _k4_body(ys_hbm, p0_hbm, p1_hbm, w01_hbm, out_hbm,
             i0_v, i1_v, w0_v, w1_v, g0_v, g1_v, sem):
    wid = lax.axis_index("s") * SC_CORES + lax.axis_index("c")
    tbase = wid * T_PER_W
    pltpu.sync_copy(p0_hbm.at[pl.ds(tbase, T_PER_W)], i0_v)
    pltpu.sync_copy(p1_hbm.at[pl.ds(tbase, T_PER_W)], i1_v)
    pltpu.sync_copy(w01_hbm.at[0].at[pl.ds(tbase, T_PER_W)], w0_v)
    pltpu.sync_copy(w01_hbm.at[1].at[pl.ds(tbase, T_PER_W)], w1_v)
    for half in range(T_PER_W // _K4_CH):
        pltpu.async_copy(
            ys_hbm.at[i0_v.at[pl.ds(half * _K4_CH, _K4_CH)]], g0_v, sem
        ).wait()
        pltpu.async_copy(
            ys_hbm.at[i1_v.at[pl.ds(half * _K4_CH, _K4_CH)]], g1_v, sem
        ).wait()

        def body(i, _, half=half):
            r = half * _K4_CH + i
            w0s = w0_v[r, :]
            w1s = w1_v[r, :]
            for col in range(D_MODEL // 16):
                sl = pl.ds(col * 16, 16)
                g0_v[i, sl] = g0_v[i, sl] * w0s + g1_v[i, sl] * w1s
            return 0

        lax.fori_loop(0, _K4_CH, body, 0)
        pltpu.sync_copy(
            g0_v, out_hbm.at[pl.ds(tbase + half * _K4_CH, _K4_CH)])


def _run_combine(ys, p0, p1, w01):
    k = functools.partial(
        pl.kernel, mesh=_sc_mesh(),
        out_type=jax.ShapeDtypeStruct((T_TOK, D_MODEL), jnp.float32),
        scratch_types=[
            pltpu.VMEM((T_PER_W,), jnp.int32),
            pltpu.VMEM((T_PER_W,), jnp.int32),
            pltpu.VMEM((T_PER_W, 16), jnp.float32),
            pltpu.VMEM((T_PER_W, 16), jnp.float32),
            pltpu.VMEM((_K4_CH, D_MODEL), jnp.float32),
            pltpu.VMEM((_K4_CH, D_MODEL), jnp.float32),
            pltpu.SemaphoreType.DMA,
        ],
    )(_k4_body)
    return k(ys, p0, p1, w01)


def kernel(x, Wg, W1, b1, W2, b2):
    Bn, Sn, D = x.shape
    E, H = W1.shape[0], W1.shape[1]
    x_flat = x.reshape(Bn * Sn, D)
    b1r = b1.reshape(E, 1, H)
    b2r = b2.reshape(E, 1, D)

    pos, w01, be24, _nu = _run_router(x_flat, Wg)
    pos3 = pos.reshape(NW, _K2_NCH, _K2_CH)
    xs = _run_scatter(x_flat, pos3)
    ys = _run_ffn(xs, W1, b1r, W2, b2r, be24.reshape(NB))
    p0 = pos[:16].reshape(T_TOK)
    p1 = pos[16:].reshape(T_TOK)
    out = _run_combine(ys, p0, p1, w01)
    return out.reshape(Bn, Sn, D)


# speedup vs baseline: 1.8236x; 1.2037x over previous
import functools

import jax
import jax.numpy as jnp
from jax import lax
from jax.experimental import pallas as pl
from jax.experimental.pallas import tpu as pltpu
from jax.experimental.pallas import tpu_sc as plsc

D_MODEL = 1024
HIDDEN = 4096
NUM_EXPERTS = 8
T_TOK = 2048

BLK = 256
NB = 2 * T_TOK // BLK + NUM_EXPERTS
PAD = NB * BLK
TH3 = 1024
HT = HIDDEN // TH3

SC_CORES = 2
SC_SUBCORES = 16
NW = SC_CORES * SC_SUBCORES
A_PER_W = 2 * T_TOK // NW
T_PER_W = T_TOK // NW


def _router_body(x_ref, wg_ref, pos_ref, w01_ref, be_ref, nu_ref):
    T = T_TOK
    E = NUM_EXPERTS
    x = x_ref[...]
    wg = wg_ref[...]
    logits = lax.dot_general(
        x, wg, (((1,), (1,)), ((), ())), preferred_element_type=jnp.float32)
    m = jnp.max(logits, axis=-1, keepdims=True)
    p = jnp.exp(logits - m)
    s = p / jnp.sum(p, axis=-1, keepdims=True)
    ei = lax.broadcasted_iota(jnp.int32, s.shape, 1)
    m1 = jnp.max(s, axis=-1, keepdims=True)
    idx1 = jnp.min(jnp.where(s == m1, ei, E), axis=-1, keepdims=True)
    oh1 = (ei == idx1).astype(jnp.float32)
    s2 = jnp.where(oh1 > 0, -jnp.inf, s)
    m2 = jnp.max(s2, axis=-1, keepdims=True)
    idx2 = jnp.min(jnp.where(s2 == m2, ei, E), axis=-1, keepdims=True)
    oh2 = (ei == idx2).astype(jnp.float32)
    denom = m1 + m2
    w0 = m1 / denom
    w1 = m2 / denom

    oh1_3 = oh1.reshape(16, 128, E)
    oh2_3 = oh2.reshape(16, 128, E)
    ii = lax.broadcasted_iota(jnp.int32, (16, 128, 128), 1)
    jj = lax.broadcasted_iota(jnp.int32, (16, 128, 128), 2)
    tril = (ii >= jj).astype(jnp.float32)
    cw1 = lax.dot_general(tril, oh1_3, (((2,), (1,)), ((0,), (0,))),
                          preferred_element_type=jnp.float32)
    cw2 = lax.dot_general(tril, oh2_3, (((2,), (1,)), ((0,), (0,))),
                          preferred_element_type=jnp.float32)
    tot1 = cw1[:, 127, :]
    tot2 = cw2[:, 127, :]
    ci = lax.broadcasted_iota(jnp.int32, (16, 16), 0)
    cj = lax.broadcasted_iota(jnp.int32, (16, 16), 1)
    l16 = (ci > cj).astype(jnp.float32)
    pre1 = lax.dot_general(l16, tot1, (((1,), (0,)), ((), ())),
                           preferred_element_type=jnp.float32)
    pre2 = lax.dot_general(l16, tot2, (((1,), (0,)), ((), ())),
                           preferred_element_type=jnp.float32)
    cex1 = cw1 + pre1.reshape(16, 1, E) - oh1_3
    cex2 = cw2 + pre2.reshape(16, 1, E) - oh2_3

    count1_row = jnp.sum(oh1, axis=0, keepdims=True)
    counts_row = count1_row + jnp.sum(oh2, axis=0, keepdims=True)
    nbk_row = jnp.floor((counts_row + (BLK - 1.0)) * (1.0 / BLK))
    e8i = lax.broadcasted_iota(jnp.int32, (E, E), 0)
    e8j = lax.broadcasted_iota(jnp.int32, (E, E), 1)
    l8t = (e8i < e8j).astype(jnp.float32)
    po_row = BLK * lax.dot_general(nbk_row, l8t, (((1,), (0,)), ((), ())),
                                   preferred_element_type=jnp.float32)
    po_b = po_row.reshape(1, 1, E)
    c1_b = count1_row.reshape(1, 1, E)

    pos0 = jnp.sum(oh1_3 * (cex1 + po_b), axis=2)
    pos1 = jnp.sum(oh2_3 * (cex2 + po_b + c1_b), axis=2)
    pos_ref[...] = jnp.concatenate(
        [pos0, pos1], axis=0).astype(jnp.int32)

    w01_ref[0] = jnp.broadcast_to(w0, (T, 16))
    w01_ref[1] = jnp.broadcast_to(w1, (T, 16))

    idm = (e8i == e8j).astype(jnp.float32)
    po_col = lax.dot_general(idm, po_row, (((0,), (1,)), ((), ())),
                             preferred_element_type=jnp.float32)
    nb24 = lax.broadcasted_iota(jnp.int32, (1, NB), 1).astype(jnp.float32)
    cmp = (po_col * (1.0 / BLK) <= nb24).astype(jnp.float32)
    be = jnp.sum(cmp, axis=0, keepdims=True) - 1.0
    ei_row = lax.broadcasted_iota(jnp.int32, (1, E), 1)
    last_e = jnp.max(jnp.where(counts_row > 0, ei_row, -1),
                     axis=1, keepdims=True).astype(jnp.float32)
    be_ref[...] = jnp.minimum(be, last_e).astype(jnp.int32)
    nu_ref[...] = jnp.sum(nbk_row, axis=1, keepdims=True).astype(jnp.int32)


def _run_router(x_flat, Wg):
    return pl.pallas_call(
        _router_body,
        out_shape=(
            jax.ShapeDtypeStruct((NW, A_PER_W), jnp.int32),
            jax.ShapeDtypeStruct((2, T_TOK, 16), jnp.float32),
            jax.ShapeDtypeStruct((1, NB), jnp.int32),
            jax.ShapeDtypeStruct((1, 1), jnp.int32),
        ),
    )(x_flat, Wg)


def _sc_mesh():
    return plsc.VectorSubcoreMesh(core_axis_name="c", subcore_axis_name="s")


_K2_CH = 32
_K2_NCH = A_PER_W // _K2_CH


def _k2_body(x_hbm, pos_hbm, xs_hbm, idx_v, rows_v, sem):
    wid = lax.axis_index("s") * SC_CORES + lax.axis_index("c")
    pltpu.sync_copy(pos_hbm.at[wid], idx_v)
    tok0 = (wid % 16) * A_PER_W
    for j in range(_K2_NCH):
        pltpu.sync_copy(x_hbm.at[pl.ds(tok0 + j * _K2_CH, _K2_CH)], rows_v)
        pltpu.async_copy(rows_v, xs_hbm.at[idx_v.at[j]], sem).wait()


def _run_scatter(x_flat, pos3):
    k = functools.partial(
        pl.kernel, mesh=_sc_mesh(),
        out_type=jax.ShapeDtypeStruct((PAD, D_MODEL), jnp.float32),
        scratch_types=[
            pltpu.VMEM((_K2_NCH, _K2_CH), jnp.int32),
            pltpu.VMEM((_K2_CH, D_MODEL), jnp.float32),
            pltpu.SemaphoreType.DMA,
        ],
    )(_k2_body)
    return k(x_flat, pos3)


HH = HIDDEN // 2


def _ffn_body(be_ref, nu_ref, xs_ref, w1_ref, b1_ref, w2_ref, b2_ref,
              ys_ref, *, add_b2):
    @pl.when(pl.program_id(0) < nu_ref[0])
    def _():
        x = xs_ref[...]
        hact = lax.dot_general(
            x, w1_ref[0], (((1,), (1,)), ((), ())),
            preferred_element_type=jnp.float32)
        hact = jnp.maximum(hact + b1_ref[0], 0.0)
        part = lax.dot_general(
            hact, w2_ref[0], (((1,), (1,)), ((), ())),
            preferred_element_type=jnp.float32)
        if add_b2:
            part = part + b2_ref[0]
        ys_ref[...] = part


def _run_ffn_half(xs, W1r, b1r, W2r, b2r, be24, nu1, hh):
    grid_spec = pltpu.PrefetchScalarGridSpec(
        num_scalar_prefetch=2,
        grid=(NB,),
        in_specs=[
            pl.BlockSpec((BLK, D_MODEL), lambda nb, be, nu: (nb, 0)),
            pl.BlockSpec((1, HH, D_MODEL),
                         lambda nb, be, nu: (be[nb], hh, 0)),
            pl.BlockSpec((1, 1, HH), lambda nb, be, nu: (be[nb], 0, hh)),
            pl.BlockSpec((1, D_MODEL, HH),
                         lambda nb, be, nu: (be[nb], 0, hh)),
            pl.BlockSpec((1, 1, D_MODEL), lambda nb, be, nu: (be[nb], 0, 0)),
        ],
        out_specs=pl.BlockSpec((BLK, D_MODEL), lambda nb, be, nu: (nb, 0)),
    )
    return pl.pallas_call(
        functools.partial(_ffn_body, add_b2=(hh == 0)),
        grid_spec=grid_spec,
        out_shape=jax.ShapeDtypeStruct((PAD, D_MODEL), jnp.float32),
        compiler_params=pltpu.CompilerParams(
            vmem_limit_bytes=60 * 1024 * 1024),
    )(be24, nu1, xs, W1r, b1r, W2r, b2r)


def _run_ffn(xs, W1, b1, W2, b2, be24, nu1):
    E = NUM_EXPERTS
    b1r = b1.reshape(E, 1, HIDDEN)
    b2r = b2.reshape(E, 1, D_MODEL)
    ys0 = _run_ffn_half(xs, W1, b1r, W2, b2r, be24, nu1, 0)
    ys1 = _run_ffn_half(xs, W1, b1r, W2, b2r, be24, nu1, 1)
    return ys0 + ys1


_K4_CH = 32


def _k4_body(ys_hbm, p0_hbm, p1_hbm, w01_hbm, out_hbm,
             i0_v, i1_v, w0_v, w1_v, g0_v, g1_v, sem):
    wid = lax.axis_index("s") * SC_CORES + lax.axis_index("c")
    tbase = wid * T_PER_W
    pltpu.sync_copy(p0_hbm.at[pl.ds(tbase, T_PER_W)], i0_v)
    pltpu.sync_copy(p1_hbm.at[pl.ds(tbase, T_PER_W)], i1_v)
    pltpu.sync_copy(w01_hbm.at[0].at[pl.ds(tbase, T_PER_W)], w0_v)
    pltpu.sync_copy(w01_hbm.at[1].at[pl.ds(tbase, T_PER_W)], w1_v)
    for half in range(T_PER_W // _K4_CH):
        pltpu.async_copy(
            ys_hbm.at[i0_v.at[pl.ds(half * _K4_CH, _K4_CH)]], g0_v, sem
        ).wait()
        pltpu.async_copy(
            ys_hbm.at[i1_v.at[pl.ds(half * _K4_CH, _K4_CH)]], g1_v, sem
        ).wait()

        def body(i, _, half=half):
            r = half * _K4_CH + i
            w0s = w0_v[r, :]
            w1s = w1_v[r, :]
            for col in range(D_MODEL // 16):
                sl = pl.ds(col * 16, 16)
                g0_v[i, sl] = g0_v[i, sl] * w0s + g1_v[i, sl] * w1s
            return 0

        lax.fori_loop(0, _K4_CH, body, 0)
        pltpu.sync_copy(
            g0_v, out_hbm.at[pl.ds(tbase + half * _K4_CH, _K4_CH)])


def _run_combine(ys, p0, p1, w01):
    k = functools.partial(
        pl.kernel, mesh=_sc_mesh(),
        out_type=jax.ShapeDtypeStruct((T_TOK, D_MODEL), jnp.float32),
        scratch_types=[
            pltpu.VMEM((T_PER_W,), jnp.int32),
            pltpu.VMEM((T_PER_W,), jnp.int32),
            pltpu.VMEM((T_PER_W, 16), jnp.float32),
            pltpu.VMEM((T_PER_W, 16), jnp.float32),
            pltpu.VMEM((_K4_CH, D_MODEL), jnp.float32),
            pltpu.VMEM((_K4_CH, D_MODEL), jnp.float32),
            pltpu.SemaphoreType.DMA,
        ],
    )(_k4_body)
    return k(ys, p0, p1, w01)


def kernel(x, Wg, W1, b1, W2, b2):
    Bn, Sn, D = x.shape
    E, H = W1.shape[0], W1.shape[1]
    x_flat = x.reshape(Bn * Sn, D)

    pos, w01, be24, nu = _run_router(x_flat, Wg)
    pos3 = pos.reshape(NW, _K2_NCH, _K2_CH)
    xs = _run_scatter(x_flat, pos3)
    ys = _run_ffn(xs, W1, b1, W2, b2, be24.reshape(NB), nu.reshape(1))
    p0 = pos[:16].reshape(T_TOK)
    p1 = pos[16:].reshape(T_TOK)
    out = _run_combine(ys, p0, p1, w01)
    return out.reshape(Bn, Sn, D)


# speedup vs baseline: 1.9921x; 1.0924x over previous
import functools

import jax
import jax.numpy as jnp
from jax import lax
from jax.experimental import pallas as pl
from jax.experimental.pallas import tpu as pltpu
from jax.experimental.pallas import tpu_sc as plsc

D_MODEL = 1024
HIDDEN = 4096
NUM_EXPERTS = 8
T_TOK = 2048

BLK = 256
NB = 2 * T_TOK // BLK + NUM_EXPERTS
PAD = NB * BLK
TH3 = 1024
HT = HIDDEN // TH3

SC_CORES = 2
SC_SUBCORES = 16
NW = SC_CORES * SC_SUBCORES
A_PER_W = 2 * T_TOK // NW
T_PER_W = T_TOK // NW


def _router_body(x_ref, wg_ref, pos_ref, w01_ref, be_ref, nu_ref):
    T = T_TOK
    E = NUM_EXPERTS
    x = x_ref[...]
    wg = wg_ref[...]
    logits = lax.dot_general(
        x, wg, (((1,), (1,)), ((), ())), preferred_element_type=jnp.float32)
    m = jnp.max(logits, axis=-1, keepdims=True)
    p = jnp.exp(logits - m)
    s = p / jnp.sum(p, axis=-1, keepdims=True)
    ei = lax.broadcasted_iota(jnp.int32, s.shape, 1)
    m1 = jnp.max(s, axis=-1, keepdims=True)
    idx1 = jnp.min(jnp.where(s == m1, ei, E), axis=-1, keepdims=True)
    oh1 = (ei == idx1).astype(jnp.float32)
    s2 = jnp.where(oh1 > 0, -jnp.inf, s)
    m2 = jnp.max(s2, axis=-1, keepdims=True)
    idx2 = jnp.min(jnp.where(s2 == m2, ei, E), axis=-1, keepdims=True)
    oh2 = (ei == idx2).astype(jnp.float32)
    denom = m1 + m2
    w0 = m1 / denom
    w1 = m2 / denom

    oh1_3 = oh1.reshape(16, 128, E)
    oh2_3 = oh2.reshape(16, 128, E)
    ii = lax.broadcasted_iota(jnp.int32, (16, 128, 128), 1)
    jj = lax.broadcasted_iota(jnp.int32, (16, 128, 128), 2)
    tril = (ii >= jj).astype(jnp.float32)
    cw1 = lax.dot_general(tril, oh1_3, (((2,), (1,)), ((0,), (0,))),
                          preferred_element_type=jnp.float32)
    cw2 = lax.dot_general(tril, oh2_3, (((2,), (1,)), ((0,), (0,))),
                          preferred_element_type=jnp.float32)
    tot1 = cw1[:, 127, :]
    tot2 = cw2[:, 127, :]
    ci = lax.broadcasted_iota(jnp.int32, (16, 16), 0)
    cj = lax.broadcasted_iota(jnp.int32, (16, 16), 1)
    l16 = (ci > cj).astype(jnp.float32)
    pre1 = lax.dot_general(l16, tot1, (((1,), (0,)), ((), ())),
                           preferred_element_type=jnp.float32)
    pre2 = lax.dot_general(l16, tot2, (((1,), (0,)), ((), ())),
                           preferred_element_type=jnp.float32)
    cex1 = cw1 + pre1.reshape(16, 1, E) - oh1_3
    cex2 = cw2 + pre2.reshape(16, 1, E) - oh2_3

    count1_row = jnp.sum(oh1, axis=0, keepdims=True)
    counts_row = count1_row + jnp.sum(oh2, axis=0, keepdims=True)
    nbk_row = jnp.floor((counts_row + (BLK - 1.0)) * (1.0 / BLK))
    e8i = lax.broadcasted_iota(jnp.int32, (E, E), 0)
    e8j = lax.broadcasted_iota(jnp.int32, (E, E), 1)
    l8t = (e8i < e8j).astype(jnp.float32)
    po_row = BLK * lax.dot_general(nbk_row, l8t, (((1,), (0,)), ((), ())),
                                   preferred_element_type=jnp.float32)
    po_b = po_row.reshape(1, 1, E)
    c1_b = count1_row.reshape(1, 1, E)

    pos0 = jnp.sum(oh1_3 * (cex1 + po_b), axis=2)
    pos1 = jnp.sum(oh2_3 * (cex2 + po_b + c1_b), axis=2)
    pos_ref[...] = jnp.concatenate(
        [pos0, pos1], axis=0).astype(jnp.int32)

    w01_ref[0] = jnp.broadcast_to(w0, (T, 16))
    w01_ref[1] = jnp.broadcast_to(w1, (T, 16))

    idm = (e8i == e8j).astype(jnp.float32)
    po_col = lax.dot_general(idm, po_row, (((0,), (1,)), ((), ())),
                             preferred_element_type=jnp.float32)
    nb24 = lax.broadcasted_iota(jnp.int32, (1, NB), 1).astype(jnp.float32)
    cmp = (po_col * (1.0 / BLK) <= nb24).astype(jnp.float32)
    be = jnp.sum(cmp, axis=0, keepdims=True) - 1.0
    ei_row = lax.broadcasted_iota(jnp.int32, (1, E), 1)
    last_e = jnp.max(jnp.where(counts_row > 0, ei_row, -1),
                     axis=1, keepdims=True).astype(jnp.float32)
    be_ref[...] = jnp.minimum(be, last_e).astype(jnp.int32)
    nu_ref[...] = jnp.sum(nbk_row, axis=1, keepdims=True).astype(jnp.int32)


def _run_router(x_flat, Wg):
    return pl.pallas_call(
        _router_body,
        out_shape=(
            jax.ShapeDtypeStruct((NW, A_PER_W), jnp.int32),
            jax.ShapeDtypeStruct((2, T_TOK, 16), jnp.float32),
            jax.ShapeDtypeStruct((1, NB), jnp.int32),
            jax.ShapeDtypeStruct((1, 1), jnp.int32),
        ),
    )(x_flat, Wg)


def _sc_mesh():
    return plsc.VectorSubcoreMesh(core_axis_name="c", subcore_axis_name="s")


_K2_CH = 32
_K2_NCH = A_PER_W // _K2_CH


def _k2_body(x_hbm, pos_hbm, xs_hbm, idx_v, rows_v, sem):
    wid = lax.axis_index("s") * SC_CORES + lax.axis_index("c")
    pltpu.sync_copy(pos_hbm.at[wid], idx_v)
    tok0 = (wid % 16) * A_PER_W
    for j in range(_K2_NCH):
        pltpu.sync_copy(x_hbm.at[pl.ds(tok0 + j * _K2_CH, _K2_CH)], rows_v)
        pltpu.async_copy(rows_v, xs_hbm.at[idx_v.at[j]], sem).wait()


def _run_scatter(x_flat, pos3):
    k = functools.partial(
        pl.kernel, mesh=_sc_mesh(),
        out_type=jax.ShapeDtypeStruct((PAD, D_MODEL), jnp.float32),
        scratch_types=[
            pltpu.VMEM((_K2_NCH, _K2_CH), jnp.int32),
            pltpu.VMEM((_K2_CH, D_MODEL), jnp.float32),
            pltpu.SemaphoreType.DMA,
        ],
    )(_k2_body)
    return k(x_flat, pos3)


HH = HIDDEN // 2


def _ffn_body(be_ref, nu_ref, xs_ref, w1_ref, b1_ref, w2_ref, b2_ref,
              *rest, add_b2):
    if add_b2:
        ys_ref = rest[0]
    else:
        ysin_ref, ys_ref = rest

    @pl.when(pl.program_id(0) < nu_ref[0])
    def _():
        x = xs_ref[...]
        hact = lax.dot_general(
            x, w1_ref[0], (((1,), (1,)), ((), ())),
            preferred_element_type=jnp.float32)
        hact = jnp.maximum(hact + b1_ref[0], 0.0)
        part = lax.dot_general(
            hact, w2_ref[0], (((1,), (1,)), ((), ())),
            preferred_element_type=jnp.float32)
        if add_b2:
            part = part + b2_ref[0]
        else:
            part = part + ysin_ref[...]
        ys_ref[...] = part


def _run_ffn_half(xs, W1r, b1r, W2r, b2r, be24, nu1, hh, ysin):
    in_specs = [
        pl.BlockSpec((BLK, D_MODEL), lambda nb, be, nu: (nb, 0)),
        pl.BlockSpec((1, HH, D_MODEL),
                     lambda nb, be, nu: (be[nb], hh, 0)),
        pl.BlockSpec((1, 1, HH), lambda nb, be, nu: (be[nb], 0, hh)),
        pl.BlockSpec((1, D_MODEL, HH),
                     lambda nb, be, nu: (be[nb], 0, hh)),
        pl.BlockSpec((1, 1, D_MODEL), lambda nb, be, nu: (be[nb], 0, 0)),
    ]
    args = [be24, nu1, xs, W1r, b1r, W2r, b2r]
    aliases = {}
    if ysin is not None:
        in_specs.append(pl.BlockSpec((BLK, D_MODEL),
                                     lambda nb, be, nu: (nb, 0)))
        args.append(ysin)
        aliases = {7: 0}
    grid_spec = pltpu.PrefetchScalarGridSpec(
        num_scalar_prefetch=2,
        grid=(NB,),
        in_specs=in_specs,
        out_specs=pl.BlockSpec((BLK, D_MODEL), lambda nb, be, nu: (nb, 0)),
    )
    return pl.pallas_call(
        functools.partial(_ffn_body, add_b2=(hh == 0)),
        grid_spec=grid_spec,
        out_shape=jax.ShapeDtypeStruct((PAD, D_MODEL), jnp.float32),
        input_output_aliases=aliases,
        compiler_params=pltpu.CompilerParams(
            vmem_limit_bytes=60 * 1024 * 1024),
    )(*args)


def _run_ffn(xs, W1, b1, W2, b2, be24, nu1):
    E = NUM_EXPERTS
    b1r = b1.reshape(E, 1, HIDDEN)
    b2r = b2.reshape(E, 1, D_MODEL)
    ys0 = _run_ffn_half(xs, W1, b1r, W2, b2r, be24, nu1, 0, None)
    return _run_ffn_half(xs, W1, b1r, W2, b2r, be24, nu1, 1, ys0)


_K4_CH = 32


def _k4_body(ys_hbm, p0_hbm, p1_hbm, w01_hbm, out_hbm,
             i0_v, i1_v, w0_v, w1_v, g0_v, g1_v, sem):
    wid = lax.axis_index("s") * SC_CORES + lax.axis_index("c")
    tbase = wid * T_PER_W
    pltpu.sync_copy(p0_hbm.at[pl.ds(tbase, T_PER_W)], i0_v)
    pltpu.sync_copy(p1_hbm.at[pl.ds(tbase, T_PER_W)], i1_v)
    pltpu.sync_copy(w01_hbm.at[0].at[pl.ds(tbase, T_PER_W)], w0_v)
    pltpu.sync_copy(w01_hbm.at[1].at[pl.ds(tbase, T_PER_W)], w1_v)
    for half in range(T_PER_W // _K4_CH):
        pltpu.async_copy(
            ys_hbm.at[i0_v.at[pl.ds(half * _K4_CH, _K4_CH)]], g0_v, sem
        ).wait()
        pltpu.async_copy(
            ys_hbm.at[i1_v.at[pl.ds(half * _K4_CH, _K4_CH)]], g1_v, sem
        ).wait()

        def body(i, _, half=half):
            r = half * _K4_CH + i
            w0s = w0_v[r, :]
            w1s = w1_v[r, :]
            for col in range(D_MODEL // 16):
                sl = pl.ds(col * 16, 16)
                g0_v[i, sl] = g0_v[i, sl] * w0s + g1_v[i, sl] * w1s
            return 0

        lax.fori_loop(0, _K4_CH, body, 0)
        pltpu.sync_copy(
            g0_v, out_hbm.at[pl.ds(tbase + half * _K4_CH, _K4_CH)])


def _run_combine(ys, p0, p1, w01):
    k = functools.partial(
        pl.kernel, mesh=_sc_mesh(),
        out_type=jax.ShapeDtypeStruct((T_TOK, D_MODEL), jnp.float32),
        scratch_types=[
            pltpu.VMEM((T_PER_W,), jnp.int32),
            pltpu.VMEM((T_PER_W,), jnp.int32),
            pltpu.VMEM((T_PER_W, 16), jnp.float32),
            pltpu.VMEM((T_PER_W, 16), jnp.float32),
            pltpu.VMEM((_K4_CH, D_MODEL), jnp.float32),
            pltpu.VMEM((_K4_CH, D_MODEL), jnp.float32),
            pltpu.SemaphoreType.DMA,
        ],
    )(_k4_body)
    return k(ys, p0, p1, w01)


def kernel(x, Wg, W1, b1, W2, b2):
    Bn, Sn, D = x.shape
    E, H = W1.shape[0], W1.shape[1]
    x_flat = x.reshape(Bn * Sn, D)

    pos, w01, be24, nu = _run_router(x_flat, Wg)
    pos3 = pos.reshape(NW, _K2_NCH, _K2_CH)
    xs = _run_scatter(x_flat, pos3)
    ys = _run_ffn(xs, W1, b1, W2, b2, be24.reshape(NB), nu.reshape(1))
    p0 = pos[:16].reshape(T_TOK)
    p1 = pos[16:].reshape(T_TOK)
    out = _run_combine(ys, p0, p1, w01)
    return out.reshape(Bn, Sn, D)


# speedup vs baseline: 2.2631x; 1.1360x over previous
import functools

import jax
import jax.numpy as jnp
from jax import lax
from jax.experimental import pallas as pl
from jax.experimental.pallas import tpu as pltpu
from jax.experimental.pallas import tpu_sc as plsc

D_MODEL = 1024
HIDDEN = 4096
NUM_EXPERTS = 8
T_TOK = 2048

BLK = 256
NB = 2 * T_TOK // BLK + NUM_EXPERTS
PAD = NB * BLK
TH3 = 1024
HT = HIDDEN // TH3

SC_CORES = 2
SC_SUBCORES = 16
NW = SC_CORES * SC_SUBCORES
A_PER_W = 2 * T_TOK // NW
T_PER_W = T_TOK // NW


def _router_body(x_ref, wg_ref, pos_ref, w01_ref, be_ref, nu_ref,
                 eo_ref, ue_ref, nue_ref):
    T = T_TOK
    E = NUM_EXPERTS
    x = x_ref[...]
    wg = wg_ref[...]
    logits = lax.dot_general(
        x, wg, (((1,), (1,)), ((), ())), preferred_element_type=jnp.float32)
    m = jnp.max(logits, axis=-1, keepdims=True)
    p = jnp.exp(logits - m)
    s = p / jnp.sum(p, axis=-1, keepdims=True)
    ei = lax.broadcasted_iota(jnp.int32, s.shape, 1)
    m1 = jnp.max(s, axis=-1, keepdims=True)
    idx1 = jnp.min(jnp.where(s == m1, ei, E), axis=-1, keepdims=True)
    oh1 = (ei == idx1).astype(jnp.float32)
    s2 = jnp.where(oh1 > 0, -jnp.inf, s)
    m2 = jnp.max(s2, axis=-1, keepdims=True)
    idx2 = jnp.min(jnp.where(s2 == m2, ei, E), axis=-1, keepdims=True)
    oh2 = (ei == idx2).astype(jnp.float32)
    denom = m1 + m2
    w0 = m1 / denom
    w1 = m2 / denom

    oh1_3 = oh1.reshape(16, 128, E)
    oh2_3 = oh2.reshape(16, 128, E)
    ii = lax.broadcasted_iota(jnp.int32, (16, 128, 128), 1)
    jj = lax.broadcasted_iota(jnp.int32, (16, 128, 128), 2)
    tril = (ii >= jj).astype(jnp.float32)
    cw1 = lax.dot_general(tril, oh1_3, (((2,), (1,)), ((0,), (0,))),
                          preferred_element_type=jnp.float32)
    cw2 = lax.dot_general(tril, oh2_3, (((2,), (1,)), ((0,), (0,))),
                          preferred_element_type=jnp.float32)
    tot1 = cw1[:, 127, :]
    tot2 = cw2[:, 127, :]
    ci = lax.broadcasted_iota(jnp.int32, (16, 16), 0)
    cj = lax.broadcasted_iota(jnp.int32, (16, 16), 1)
    l16 = (ci > cj).astype(jnp.float32)
    pre1 = lax.dot_general(l16, tot1, (((1,), (0,)), ((), ())),
                           preferred_element_type=jnp.float32)
    pre2 = lax.dot_general(l16, tot2, (((1,), (0,)), ((), ())),
                           preferred_element_type=jnp.float32)
    cex1 = cw1 + pre1.reshape(16, 1, E) - oh1_3
    cex2 = cw2 + pre2.reshape(16, 1, E) - oh2_3

    count1_row = jnp.sum(oh1, axis=0, keepdims=True)
    counts_row = count1_row + jnp.sum(oh2, axis=0, keepdims=True)
    nbk_row = jnp.floor((counts_row + (BLK - 1.0)) * (1.0 / BLK))
    e8i = lax.broadcasted_iota(jnp.int32, (E, E), 0)
    e8j = lax.broadcasted_iota(jnp.int32, (E, E), 1)
    l8t = (e8i < e8j).astype(jnp.float32)
    po_row = BLK * lax.dot_general(nbk_row, l8t, (((1,), (0,)), ((), ())),
                                   preferred_element_type=jnp.float32)
    po_b = po_row.reshape(1, 1, E)
    c1_b = count1_row.reshape(1, 1, E)

    pos0 = jnp.sum(oh1_3 * (cex1 + po_b), axis=2)
    pos1 = jnp.sum(oh2_3 * (cex2 + po_b + c1_b), axis=2)
    pos_ref[...] = jnp.concatenate(
        [pos0, pos1], axis=0).astype(jnp.int32)

    w01_ref[0] = jnp.broadcast_to(w0, (T, 16))
    w01_ref[1] = jnp.broadcast_to(w1, (T, 16))

    idm = (e8i == e8j).astype(jnp.float32)
    po_col = lax.dot_general(idm, po_row, (((0,), (1,)), ((), ())),
                             preferred_element_type=jnp.float32)
    nb24 = lax.broadcasted_iota(jnp.int32, (1, NB), 1).astype(jnp.float32)
    cmp = (po_col * (1.0 / BLK) <= nb24).astype(jnp.float32)
    be = jnp.sum(cmp, axis=0, keepdims=True) - 1.0
    ei_row = lax.broadcasted_iota(jnp.int32, (1, E), 1)
    last_e = jnp.max(jnp.where(counts_row > 0, ei_row, -1),
                     axis=1, keepdims=True).astype(jnp.float32)
    be_f = jnp.minimum(be, last_e)
    be_ref[...] = be_f.astype(jnp.int32)
    nu_ref[...] = jnp.sum(nbk_row, axis=1, keepdims=True).astype(jnp.int32)

    used_row = (counts_row > 0).astype(jnp.float32)
    counts_col = lax.dot_general(idm, counts_row, (((0,), (1,)), ((), ())),
                                 preferred_element_type=jnp.float32)
    used_col = (counts_col > 0).astype(jnp.float32)
    nue_ref[...] = jnp.sum(used_row, axis=1,
                           keepdims=True).astype(jnp.int32)
    rank_row = lax.dot_general(used_row, l8t, (((1,), (0,)), ((), ())),
                               preferred_element_type=jnp.float32)
    o_col = lax.broadcasted_iota(jnp.int32, (E, 1), 0).astype(jnp.float32)
    sel = (o_col == rank_row).astype(jnp.float32) * used_row
    ue_ref[...] = lax.dot_general(
        sel, o_col, (((1,), (0,)), ((), ())),
        preferred_element_type=jnp.float32).astype(jnp.int32)
    cmp_eo = (o_col < be_f).astype(jnp.float32) * used_col
    eo_ref[...] = jnp.sum(cmp_eo, axis=0, keepdims=True).astype(jnp.int32)


def _run_router(x_flat, Wg):
    return pl.pallas_call(
        _router_body,
        out_shape=(
            jax.ShapeDtypeStruct((NW, A_PER_W), jnp.int32),
            jax.ShapeDtypeStruct((2, T_TOK, 16), jnp.float32),
            jax.ShapeDtypeStruct((1, NB), jnp.int32),
            jax.ShapeDtypeStruct((1, 1), jnp.int32),
            jax.ShapeDtypeStruct((1, NB), jnp.int32),
            jax.ShapeDtypeStruct((NUM_EXPERTS, 1), jnp.int32),
            jax.ShapeDtypeStruct((1, 1), jnp.int32),
        ),
    )(x_flat, Wg)


def _sc_mesh():
    return plsc.VectorSubcoreMesh(core_axis_name="c", subcore_axis_name="s")


_K2_CH = 32
_K2_NCH = A_PER_W // _K2_CH


def _k2_body(x_hbm, pos_hbm, xs_hbm, idx_v, rows_v, sem):
    wid = lax.axis_index("s") * SC_CORES + lax.axis_index("c")
    pltpu.sync_copy(pos_hbm.at[wid], idx_v)
    tok0 = (wid % 16) * A_PER_W
    for j in range(_K2_NCH):
        pltpu.sync_copy(x_hbm.at[pl.ds(tok0 + j * _K2_CH, _K2_CH)], rows_v)
        pltpu.async_copy(rows_v, xs_hbm.at[idx_v.at[j]], sem).wait()


def _run_scatter(x_flat, pos3):
    k = functools.partial(
        pl.kernel, mesh=_sc_mesh(),
        out_type=jax.ShapeDtypeStruct((PAD, D_MODEL), jnp.float32),
        scratch_types=[
            pltpu.VMEM((_K2_NCH, _K2_CH), jnp.int32),
            pltpu.VMEM((_K2_CH, D_MODEL), jnp.float32),
            pltpu.SemaphoreType.DMA,
        ],
    )(_k2_body)
    return k(x_flat, pos3)


HH = HIDDEN // 2


def _ffn_body(be_ref, nu_ref, eo_ref, ue_ref, nue_ref,
              xs_ref, w1_ref, b1_ref, w2_ref, b2_ref,
              *rest, add_b2, hh):
    if add_b2:
        ys_ref, w1s_ref, w2s_ref, sems = rest
    else:
        ysin_ref, ys_ref, w1s_ref, w2s_ref, sems = rest

    nb = pl.program_id(0)
    active = nb < nu_ref[0]
    eo = eo_ref[nb]
    slot = lax.rem(eo, 2)
    prev_eo = eo_ref[jnp.maximum(nb - 1, 0)]
    fresh = jnp.logical_or(nb == 0, eo != prev_eo)
    nue = nue_ref[0]

    def _w_dma(o, s):
        e = ue_ref[o]
        return (pltpu.make_async_copy(
                    w1_ref.at[e, pl.ds(hh * HH, HH), :], w1s_ref.at[s],
                    sems.at[s, 0]),
                pltpu.make_async_copy(
                    w2_ref.at[e, :, pl.ds(hh * HH, HH)], w2s_ref.at[s],
                    sems.at[s, 1]))

    @pl.when(nb == 0)
    def _prime():
        d1, d2 = _w_dma(0, 0)
        d1.start()
        d2.start()

        @pl.when(nue > 1)
        def _():
            d1b, d2b = _w_dma(1, 1)
            d1b.start()
            d2b.start()

    @pl.when(jnp.logical_and(active, fresh))
    def _rotate():
        d1, d2 = _w_dma(eo, slot)
        d1.wait()
        d2.wait()

        @pl.when(jnp.logical_and(nb > 0, eo + 1 < nue))
        def _():
            d1n, d2n = _w_dma(eo + 1, 1 - slot)
            d1n.start()
            d2n.start()

    @pl.when(active)
    def _():
        x = xs_ref[...]
        hact = lax.dot_general(
            x, w1s_ref[slot], (((1,), (1,)), ((), ())),
            preferred_element_type=jnp.float32)
        hact = jnp.maximum(hact + b1_ref[0], 0.0)
        part = lax.dot_general(
            hact, w2s_ref[slot], (((1,), (1,)), ((), ())),
            preferred_element_type=jnp.float32)
        if add_b2:
            part = part + b2_ref[0]
        else:
            part = part + ysin_ref[...]
        ys_ref[...] = part


def _run_ffn_half(xs, W1r, b1r, W2r, b2r, scal, hh, ysin):
    in_specs = [
        pl.BlockSpec((BLK, D_MODEL), lambda nb, *s: (nb, 0)),
        pl.BlockSpec(memory_space=pl.ANY),
        pl.BlockSpec((1, 1, HH), lambda nb, be, *s: (be[nb], 0, hh)),
        pl.BlockSpec(memory_space=pl.ANY),
        pl.BlockSpec((1, 1, D_MODEL), lambda nb, be, *s: (be[nb], 0, 0)),
    ]
    args = list(scal) + [xs, W1r, b1r, W2r, b2r]
    aliases = {}
    if ysin is not None:
        in_specs.append(pl.BlockSpec((BLK, D_MODEL), lambda nb, *s: (nb, 0)))
        args.append(ysin)
        aliases = {len(args) - 1: 0}
    grid_spec = pltpu.PrefetchScalarGridSpec(
        num_scalar_prefetch=5,
        grid=(NB,),
        in_specs=in_specs,
        out_specs=pl.BlockSpec((BLK, D_MODEL), lambda nb, *s: (nb, 0)),
        scratch_shapes=[
            pltpu.VMEM((2, HH, D_MODEL), jnp.float32),
            pltpu.VMEM((2, D_MODEL, HH), jnp.float32),
            pltpu.SemaphoreType.DMA((2, 2)),
        ],
    )
    return pl.pallas_call(
        functools.partial(_ffn_body, add_b2=(hh == 0), hh=hh),
        grid_spec=grid_spec,
        out_shape=jax.ShapeDtypeStruct((PAD, D_MODEL), jnp.float32),
        input_output_aliases=aliases,
        compiler_params=pltpu.CompilerParams(
            vmem_limit_bytes=60 * 1024 * 1024),
    )(*args)


def _run_ffn(xs, W1, b1, W2, b2, scal):
    E = NUM_EXPERTS
    b1r = b1.reshape(E, 1, HIDDEN)
    b2r = b2.reshape(E, 1, D_MODEL)
    ys0 = _run_ffn_half(xs, W1, b1r, W2, b2r, scal, 0, None)
    return _run_ffn_half(xs, W1, b1r, W2, b2r, scal, 1, ys0)


_K4_CH = 32


def _k4_body(ys_hbm, p0_hbm, p1_hbm, w01_hbm, out_hbm,
             i0_v, i1_v, w0_v, w1_v, g0_v, g1_v, sem):
    wid = lax.axis_index("s") * SC_CORES + lax.axis_index("c")
    tbase = wid * T_PER_W
    pltpu.sync_copy(p0_hbm.at[pl.ds(tbase, T_PER_W)], i0_v)
    pltpu.sync_copy(p1_hbm.at[pl.ds(tbase, T_PER_W)], i1_v)
    pltpu.sync_copy(w01_hbm.at[0].at[pl.ds(tbase, T_PER_W)], w0_v)
    pltpu.sync_copy(w01_hbm.at[1].at[pl.ds(tbase, T_PER_W)], w1_v)
    for half in range(T_PER_W // _K4_CH):
        pltpu.async_copy(
            ys_hbm.at[i0_v.at[pl.ds(half * _K4_CH, _K4_CH)]], g0_v, sem
        ).wait()
        pltpu.async_copy(
            ys_hbm.at[i1_v.at[pl.ds(half * _K4_CH, _K4_CH)]], g1_v, sem
        ).wait()

        def body(i, _, half=half):
            r = half * _K4_CH + i
            w0s = w0_v[r, :]
            w1s = w1_v[r, :]
            for col in range(D_MODEL // 16):
                sl = pl.ds(col * 16, 16)
                g0_v[i, sl] = g0_v[i, sl] * w0s + g1_v[i, sl] * w1s
            return 0

        lax.fori_loop(0, _K4_CH, body, 0)
        pltpu.sync_copy(
            g0_v, out_hbm.at[pl.ds(tbase + half * _K4_CH, _K4_CH)])


def _run_combine(ys, p0, p1, w01):
    k = functools.partial(
        pl.kernel, mesh=_sc_mesh(),
        out_type=jax.ShapeDtypeStruct((T_TOK, D_MODEL), jnp.float32),
        scratch_types=[
            pltpu.VMEM((T_PER_W,), jnp.int32),
            pltpu.VMEM((T_PER_W,), jnp.int32),
            pltpu.VMEM((T_PER_W, 16), jnp.float32),
            pltpu.VMEM((T_PER_W, 16), jnp.float32),
            pltpu.VMEM((_K4_CH, D_MODEL), jnp.float32),
            pltpu.VMEM((_K4_CH, D_MODEL), jnp.float32),
            pltpu.SemaphoreType.DMA,
        ],
    )(_k4_body)
    return k(ys, p0, p1, w01)


def kernel(x, Wg, W1, b1, W2, b2):
    Bn, Sn, D = x.shape
    E, H = W1.shape[0], W1.shape[1]
    x_flat = x.reshape(Bn * Sn, D)

    pos, w01, be24, nu, eo24, ue8, nue = _run_router(x_flat, Wg)
    pos3 = pos.reshape(NW, _K2_NCH, _K2_CH)
    xs = _run_scatter(x_flat, pos3)
    scal = (be24.reshape(NB), nu.reshape(1), eo24.reshape(NB),
            ue8.reshape(NUM_EXPERTS), nue.reshape(1))
    ys = _run_ffn(xs, W1, b1, W2, b2, scal)
    p0 = pos[:16].reshape(T_TOK)
    p1 = pos[16:].reshape(T_TOK)
    out = _run_combine(ys, p0, p1, w01)
    return out.reshape(Bn, Sn, D)


# speedup vs baseline: 2.2888x; 1.0114x over previous
import functools

import jax
import jax.numpy as jnp
from jax import lax
from jax.experimental import pallas as pl
from jax.experimental.pallas import tpu as pltpu
from jax.experimental.pallas import tpu_sc as plsc

D_MODEL = 1024
HIDDEN = 4096
NUM_EXPERTS = 8
T_TOK = 2048

BLK = 256
NB = 2 * T_TOK // BLK + NUM_EXPERTS
PAD = NB * BLK
TH3 = 1024
HT = HIDDEN // TH3

SC_CORES = 2
SC_SUBCORES = 16
NW = SC_CORES * SC_SUBCORES
A_PER_W = 2 * T_TOK // NW
T_PER_W = T_TOK // NW


def _router_body(x_ref, wg_ref, pos_ref, w01_ref, be_ref, nu_ref,
                 eo_ref, ue_ref, nue_ref):
    T = T_TOK
    E = NUM_EXPERTS
    x = x_ref[...]
    wg = wg_ref[...]
    logits = lax.dot_general(
        x, wg, (((1,), (1,)), ((), ())), preferred_element_type=jnp.float32)
    m = jnp.max(logits, axis=-1, keepdims=True)
    p = jnp.exp(logits - m)
    s = p / jnp.sum(p, axis=-1, keepdims=True)
    ei = lax.broadcasted_iota(jnp.int32, s.shape, 1)
    m1 = jnp.max(s, axis=-1, keepdims=True)
    idx1 = jnp.min(jnp.where(s == m1, ei, E), axis=-1, keepdims=True)
    oh1 = (ei == idx1).astype(jnp.float32)
    s2 = jnp.where(oh1 > 0, -jnp.inf, s)
    m2 = jnp.max(s2, axis=-1, keepdims=True)
    idx2 = jnp.min(jnp.where(s2 == m2, ei, E), axis=-1, keepdims=True)
    oh2 = (ei == idx2).astype(jnp.float32)
    denom = m1 + m2
    w0 = m1 / denom
    w1 = m2 / denom

    oh1_3 = oh1.reshape(16, 128, E)
    oh2_3 = oh2.reshape(16, 128, E)
    ii = lax.broadcasted_iota(jnp.int32, (16, 128, 128), 1)
    jj = lax.broadcasted_iota(jnp.int32, (16, 128, 128), 2)
    tril = (ii >= jj).astype(jnp.float32)
    cw1 = lax.dot_general(tril, oh1_3, (((2,), (1,)), ((0,), (0,))),
                          preferred_element_type=jnp.float32)
    cw2 = lax.dot_general(tril, oh2_3, (((2,), (1,)), ((0,), (0,))),
                          preferred_element_type=jnp.float32)
    tot1 = cw1[:, 127, :]
    tot2 = cw2[:, 127, :]
    ci = lax.broadcasted_iota(jnp.int32, (16, 16), 0)
    cj = lax.broadcasted_iota(jnp.int32, (16, 16), 1)
    l16 = (ci > cj).astype(jnp.float32)
    pre1 = lax.dot_general(l16, tot1, (((1,), (0,)), ((), ())),
                           preferred_element_type=jnp.float32)
    pre2 = lax.dot_general(l16, tot2, (((1,), (0,)), ((), ())),
                           preferred_element_type=jnp.float32)
    cex1 = cw1 + pre1.reshape(16, 1, E) - oh1_3
    cex2 = cw2 + pre2.reshape(16, 1, E) - oh2_3

    count1_row = jnp.sum(oh1, axis=0, keepdims=True)
    counts_row = count1_row + jnp.sum(oh2, axis=0, keepdims=True)
    nbk_row = jnp.floor((counts_row + (BLK - 1.0)) * (1.0 / BLK))
    e8i = lax.broadcasted_iota(jnp.int32, (E, E), 0)
    e8j = lax.broadcasted_iota(jnp.int32, (E, E), 1)
    l8t = (e8i < e8j).astype(jnp.float32)
    po_row = BLK * lax.dot_general(nbk_row, l8t, (((1,), (0,)), ((), ())),
                                   preferred_element_type=jnp.float32)
    po_b = po_row.reshape(1, 1, E)
    c1_b = count1_row.reshape(1, 1, E)

    pos0 = jnp.sum(oh1_3 * (cex1 + po_b), axis=2)
    pos1 = jnp.sum(oh2_3 * (cex2 + po_b + c1_b), axis=2)
    pos_ref[...] = jnp.concatenate(
        [pos0, pos1], axis=0).astype(jnp.int32)

    w01_ref[0] = jnp.broadcast_to(w0, (T, 16))
    w01_ref[1] = jnp.broadcast_to(w1, (T, 16))

    idm = (e8i == e8j).astype(jnp.float32)
    po_col = lax.dot_general(idm, po_row, (((0,), (1,)), ((), ())),
                             preferred_element_type=jnp.float32)
    nb24 = lax.broadcasted_iota(jnp.int32, (1, NB), 1).astype(jnp.float32)
    cmp = (po_col * (1.0 / BLK) <= nb24).astype(jnp.float32)
    be = jnp.sum(cmp, axis=0, keepdims=True) - 1.0
    ei_row = lax.broadcasted_iota(jnp.int32, (1, E), 1)
    last_e = jnp.max(jnp.where(counts_row > 0, ei_row, -1),
                     axis=1, keepdims=True).astype(jnp.float32)
    be_f = jnp.minimum(be, last_e)
    be_ref[...] = be_f.astype(jnp.int32)
    nu_ref[...] = jnp.sum(nbk_row, axis=1, keepdims=True).astype(jnp.int32)

    used_row = (counts_row > 0).astype(jnp.float32)
    counts_col = lax.dot_general(idm, counts_row, (((0,), (1,)), ((), ())),
                                 preferred_element_type=jnp.float32)
    used_col = (counts_col > 0).astype(jnp.float32)
    nue_ref[...] = jnp.sum(used_row, axis=1,
                           keepdims=True).astype(jnp.int32)
    rank_row = lax.dot_general(used_row, l8t, (((1,), (0,)), ((), ())),
                               preferred_element_type=jnp.float32)
    o_col = lax.broadcasted_iota(jnp.int32, (E, 1), 0).astype(jnp.float32)
    sel = (o_col == rank_row).astype(jnp.float32) * used_row
    ue_ref[...] = lax.dot_general(
        sel, o_col, (((1,), (0,)), ((), ())),
        preferred_element_type=jnp.float32).astype(jnp.int32)
    cmp_eo = (o_col < be_f).astype(jnp.float32) * used_col
    eo_ref[...] = jnp.sum(cmp_eo, axis=0, keepdims=True).astype(jnp.int32)


def _run_router(x_flat, Wg):
    return pl.pallas_call(
        _router_body,
        out_shape=(
            jax.ShapeDtypeStruct((NW, A_PER_W), jnp.int32),
            jax.ShapeDtypeStruct((2, T_TOK, 16), jnp.float32),
            jax.ShapeDtypeStruct((1, NB), jnp.int32),
            jax.ShapeDtypeStruct((1, 1), jnp.int32),
            jax.ShapeDtypeStruct((1, NB), jnp.int32),
            jax.ShapeDtypeStruct((NUM_EXPERTS, 1), jnp.int32),
            jax.ShapeDtypeStruct((1, 1), jnp.int32),
        ),
    )(x_flat, Wg)


def _sc_mesh():
    return plsc.VectorSubcoreMesh(core_axis_name="c", subcore_axis_name="s")


_K2_CH = 32
_K2_NCH = A_PER_W // _K2_CH


def _k2_body(x_hbm, pos_hbm, xs_hbm, idx_v, rows_v, lsem, ssem):
    wid = lax.axis_index("s") * SC_CORES + lax.axis_index("c")
    pltpu.sync_copy(pos_hbm.at[wid], idx_v)
    tok0 = (wid % 16) * A_PER_W

    def ld(c, s):
        return pltpu.make_async_copy(
            x_hbm.at[pl.ds(tok0 + c * _K2_CH, _K2_CH)], rows_v.at[s],
            lsem.at[s])

    def sc(c, s):
        return pltpu.make_async_copy(
            rows_v.at[s], xs_hbm.at[idx_v.at[c]], ssem.at[s])

    ld(0, 0).start()
    for c in range(_K2_NCH):
        s = c % 2
        ld(c, s).wait()
        if c >= 1:
            sc(c - 1, 1 - s).wait()
        if c + 1 < _K2_NCH:
            ld(c + 1, 1 - s).start()
        sc(c, s).start()
    sc(_K2_NCH - 1, (_K2_NCH - 1) % 2).wait()


def _run_scatter(x_flat, pos3):
    k = functools.partial(
        pl.kernel, mesh=_sc_mesh(),
        out_type=jax.ShapeDtypeStruct((PAD, D_MODEL), jnp.float32),
        scratch_types=[
            pltpu.VMEM((_K2_NCH, _K2_CH), jnp.int32),
            pltpu.VMEM((2, _K2_CH, D_MODEL), jnp.float32),
            pltpu.SemaphoreType.DMA((2,)),
            pltpu.SemaphoreType.DMA((2,)),
        ],
    )(_k2_body)
    return k(x_flat, pos3)


HH = HIDDEN // 2


def _ffn_body(be_ref, nu_ref, eo_ref, ue_ref, nue_ref,
              xs_ref, w1_ref, b1_ref, w2_ref, b2_ref,
              *rest, add_b2, hh):
    if add_b2:
        ys_ref, w1s_ref, w2s_ref, sems = rest
    else:
        ysin_ref, ys_ref, w1s_ref, w2s_ref, sems = rest

    nb = pl.program_id(0)
    active = nb < nu_ref[0]
    eo = eo_ref[nb]
    slot = lax.rem(eo, 2)
    prev_eo = eo_ref[jnp.maximum(nb - 1, 0)]
    fresh = jnp.logical_or(nb == 0, eo != prev_eo)
    nue = nue_ref[0]

    def _w_dma(o, s):
        e = ue_ref[o]
        return (pltpu.make_async_copy(
                    w1_ref.at[e, pl.ds(hh * HH, HH), :], w1s_ref.at[s],
                    sems.at[s, 0]),
                pltpu.make_async_copy(
                    w2_ref.at[e, :, pl.ds(hh * HH, HH)], w2s_ref.at[s],
                    sems.at[s, 1]))

    @pl.when(nb == 0)
    def _prime():
        d1, d2 = _w_dma(0, 0)
        d1.start()
        d2.start()

        @pl.when(nue > 1)
        def _():
            d1b, d2b = _w_dma(1, 1)
            d1b.start()
            d2b.start()

    @pl.when(jnp.logical_and(active, fresh))
    def _rotate():
        d1, d2 = _w_dma(eo, slot)
        d1.wait()
        d2.wait()

        @pl.when(jnp.logical_and(nb > 0, eo + 1 < nue))
        def _():
            d1n, d2n = _w_dma(eo + 1, 1 - slot)
            d1n.start()
            d2n.start()

    @pl.when(active)
    def _():
        x = xs_ref[...]
        hact = lax.dot_general(
            x, w1s_ref[slot], (((1,), (1,)), ((), ())),
            preferred_element_type=jnp.float32)
        hact = jnp.maximum(hact + b1_ref[0], 0.0)
        part = lax.dot_general(
            hact, w2s_ref[slot], (((1,), (1,)), ((), ())),
            preferred_element_type=jnp.float32)
        if add_b2:
            part = part + b2_ref[0]
        else:
            part = part + ysin_ref[...]
        ys_ref[...] = part


def _run_ffn_half(xs, W1r, b1r, W2r, b2r, scal, hh, ysin):
    in_specs = [
        pl.BlockSpec((BLK, D_MODEL), lambda nb, *s: (nb, 0)),
        pl.BlockSpec(memory_space=pl.ANY),
        pl.BlockSpec((1, 1, HH), lambda nb, be, *s: (be[nb], 0, hh)),
        pl.BlockSpec(memory_space=pl.ANY),
        pl.BlockSpec((1, 1, D_MODEL), lambda nb, be, *s: (be[nb], 0, 0)),
    ]
    args = list(scal) + [xs, W1r, b1r, W2r, b2r]
    aliases = {}
    if ysin is not None:
        in_specs.append(pl.BlockSpec((BLK, D_MODEL), lambda nb, *s: (nb, 0)))
        args.append(ysin)
        aliases = {len(args) - 1: 0}
    grid_spec = pltpu.PrefetchScalarGridSpec(
        num_scalar_prefetch=5,
        grid=(NB,),
        in_specs=in_specs,
        out_specs=pl.BlockSpec((BLK, D_MODEL), lambda nb, *s: (nb, 0)),
        scratch_shapes=[
            pltpu.VMEM((2, HH, D_MODEL), jnp.float32),
            pltpu.VMEM((2, D_MODEL, HH), jnp.float32),
            pltpu.SemaphoreType.DMA((2, 2)),
        ],
    )
    return pl.pallas_call(
        functools.partial(_ffn_body, add_b2=(hh == 0), hh=hh),
        grid_spec=grid_spec,
        out_shape=jax.ShapeDtypeStruct((PAD, D_MODEL), jnp.float32),
        input_output_aliases=aliases,
        compiler_params=pltpu.CompilerParams(
            vmem_limit_bytes=60 * 1024 * 1024),
    )(*args)


def _run_ffn(xs, W1, b1, W2, b2, scal):
    E = NUM_EXPERTS
    b1r = b1.reshape(E, 1, HIDDEN)
    b2r = b2.reshape(E, 1, D_MODEL)
    ys0 = _run_ffn_half(xs, W1, b1r, W2, b2r, scal, 0, None)
    return _run_ffn_half(xs, W1, b1r, W2, b2r, scal, 1, ys0)


_K4_CH = 16


_K4_NCH = T_PER_W // _K4_CH


def _k4_body(ys_hbm, p0_hbm, p1_hbm, w01_hbm, out_hbm,
             i0_v, i1_v, w0_v, w1_v, g0_v, g1_v, ob_v, gsem, wsem):
    wid = lax.axis_index("s") * SC_CORES + lax.axis_index("c")
    tbase = wid * T_PER_W
    pltpu.sync_copy(p0_hbm.at[pl.ds(tbase, T_PER_W)], i0_v)
    pltpu.sync_copy(p1_hbm.at[pl.ds(tbase, T_PER_W)], i1_v)
    pltpu.sync_copy(w01_hbm.at[0].at[pl.ds(tbase, T_PER_W)], w0_v)
    pltpu.sync_copy(w01_hbm.at[1].at[pl.ds(tbase, T_PER_W)], w1_v)

    def gat(c, s):
        sl = pl.ds(c * _K4_CH, _K4_CH)
        return (pltpu.make_async_copy(ys_hbm.at[i0_v.at[sl]], g0_v.at[s],
                                      gsem.at[s, 0]),
                pltpu.make_async_copy(ys_hbm.at[i1_v.at[sl]], g1_v.at[s],
                                      gsem.at[s, 1]))

    def wb(c, s):
        return pltpu.make_async_copy(
            ob_v.at[s], out_hbm.at[pl.ds(tbase + c * _K4_CH, _K4_CH)],
            wsem.at[s])

    a0, b0 = gat(0, 0)
    a0.start()
    b0.start()
    for c in range(_K4_NCH):
        s = c % 2
        if c + 1 < _K4_NCH:
            a, b = gat(c + 1, 1 - s)
            a.start()
            b.start()
        ga, gb = gat(c, s)
        ga.wait()
        gb.wait()
        if c >= 2:
            wb(c - 2, s).wait()

        def body(i, _, s=s, c=c):
            r = c * _K4_CH + i
            w0s = w0_v[r, :]
            w1s = w1_v[r, :]
            for col in range(D_MODEL // 16):
                sl = pl.ds(col * 16, 16)
                ob_v[s, i, sl] = g0_v[s, i, sl] * w0s + g1_v[s, i, sl] * w1s
            return 0

        lax.fori_loop(0, _K4_CH, body, 0)
        wb(c, s).start()
    wb(_K4_NCH - 2, _K4_NCH % 2).wait()
    wb(_K4_NCH - 1, 1 - _K4_NCH % 2).wait()


def _run_combine(ys, p0, p1, w01):
    k = functools.partial(
        pl.kernel, mesh=_sc_mesh(),
        out_type=jax.ShapeDtypeStruct((T_TOK, D_MODEL), jnp.float32),
        scratch_types=[
            pltpu.VMEM((T_PER_W,), jnp.int32),
            pltpu.VMEM((T_PER_W,), jnp.int32),
            pltpu.VMEM((T_PER_W, 16), jnp.float32),
            pltpu.VMEM((T_PER_W, 16), jnp.float32),
            pltpu.VMEM((2, _K4_CH, D_MODEL), jnp.float32),
            pltpu.VMEM((2, _K4_CH, D_MODEL), jnp.float32),
            pltpu.VMEM((2, _K4_CH, D_MODEL), jnp.float32),
            pltpu.SemaphoreType.DMA((2, 2)),
            pltpu.SemaphoreType.DMA((2,)),
        ],
    )(_k4_body)
    return k(ys, p0, p1, w01)


def kernel(x, Wg, W1, b1, W2, b2):
    Bn, Sn, D = x.shape
    E, H = W1.shape[0], W1.shape[1]
    x_flat = x.reshape(Bn * Sn, D)

    pos, w01, be24, nu, eo24, ue8, nue = _run_router(x_flat, Wg)
    pos3 = pos.reshape(NW, _K2_NCH, _K2_CH)
    xs = _run_scatter(x_flat, pos3)
    scal = (be24.reshape(NB), nu.reshape(1), eo24.reshape(NB),
            ue8.reshape(NUM_EXPERTS), nue.reshape(1))
    ys = _run_ffn(xs, W1, b1, W2, b2, scal)
    p0 = pos[:16].reshape(T_TOK)
    p1 = pos[16:].reshape(T_TOK)
    out = _run_combine(ys, p0, p1, w01)
    return out.reshape(Bn, Sn, D)
